# baseline (speedup 1.0000x reference)
import functools

import jax
import jax.numpy as jnp
from jax import lax
from jax.experimental import pallas as pl
from jax.experimental.pallas import tpu as pltpu
from jax.experimental.pallas import tpu_sc as plsc

NUM_LAYERS = 2
POOL_SIZE = 1000
LENGTH = 5
NUM_HEADS = 12
EMBED_DIM = 768
HEAD_DIM = EMBED_DIM // NUM_HEADS
TOP_K = 4
BATCH = 128
SEQ = 197

ROW_D = LENGTH * EMBED_DIM
TABLE_ROWS = NUM_LAYERS * 2 * POOL_SIZE
OUT_ROWS = NUM_LAYERS * 2 * BATCH * TOP_K
NUM_WORKERS = 32
ROWS_PER_W = OUT_ROWS // NUM_WORKERS
CHUNK = 16
NCHUNK = ROWS_PER_W // CHUNK

XB = 8


def _xnorm_body(x_ref, o_ref):
    x = x_ref[...]
    m = jnp.mean(x, axis=1)
    ss = jnp.sum(m * m, axis=-1, keepdims=True)
    o_ref[...] = m * lax.rsqrt(jnp.maximum(ss, 1e-12))


def _topk_body(xn_ref, key_ref, idx_ref, rs_ref):
    xn = xn_ref[...]
    key = key_ref[...]
    ss = jnp.sum(key * key, axis=-1, keepdims=True)
    keyn = key * lax.rsqrt(jnp.maximum(ss, 1e-12))
    sim = lax.dot_general(xn, keyn, (((1,), (1,)), ((), ())),
                          preferred_element_type=jnp.float32)
    iota = lax.broadcasted_iota(jnp.int32, sim.shape, 1)
    total = jnp.float32(0.0)
    cols = []
    for _ in range(TOP_K):
        m = jnp.max(sim, axis=1, keepdims=True)
        am = jnp.min(jnp.where(sim == m, iota, jnp.int32(2**30)),
                     axis=1, keepdims=True)
        cols.append(am)
        total = total + jnp.sum(m)
        sim = jnp.where(iota == am, -jnp.inf, sim)
    idx_ref[...] = jnp.concatenate(cols, axis=1)
    rs_ref[...] = jnp.full((1, 1), total / BATCH, jnp.float32)


@functools.cache
def _make_sc_gather():
    mesh = plsc.VectorSubcoreMesh(core_axis_name="c", subcore_axis_name="s")

    @functools.partial(
        pl.kernel,
        mesh=mesh,
        out_type=jax.ShapeDtypeStruct((OUT_ROWS, ROW_D), jnp.float32),
        scratch_types=[
            pltpu.VMEM((NCHUNK, CHUNK), jnp.int32),
            pltpu.VMEM((CHUNK, ROW_D), jnp.float32),
            pltpu.VMEM((CHUNK, ROW_D), jnp.float32),
            pltpu.SemaphoreType.DMA,
            pltpu.SemaphoreType.DMA,
        ],
    )
    def _sc_gather(table_hbm, g_hbm, out_hbm, idx_v, buf0, buf1, sem0, sem1):
        wid = lax.axis_index("s") * 2 + lax.axis_index("c")
        base = wid * ROWS_PER_W
        pltpu.sync_copy(g_hbm.at[wid], idx_v)
        bufs = (buf0, buf1)
        sems = (sem0, sem1)
        handles = [None, None]
        handles[0] = pltpu.async_copy(table_hbm.at[idx_v.at[0]], bufs[0], sems[0])
        for c in range(NCHUNK):
            nxt = c + 1
            if nxt < NCHUNK:
                handles[nxt % 2] = pltpu.async_copy(
                    table_hbm.at[idx_v.at[nxt]], bufs[nxt % 2], sems[nxt % 2])
            handles[c % 2].wait()
            pltpu.sync_copy(bufs[c % 2],
                            out_hbm.at[pl.ds(base + c * CHUNK, CHUNK)])

    return _sc_gather


def kernel(x_embed, prompt, prompt_key):
    x_norm = pl.pallas_call(
        _xnorm_body,
        grid=(BATCH // XB,),
        in_specs=[pl.BlockSpec((XB, SEQ, EMBED_DIM), lambda i: (i, 0, 0))],
        out_specs=pl.BlockSpec((XB, EMBED_DIM), lambda i: (i, 0)),
        out_shape=jax.ShapeDtypeStruct((BATCH, EMBED_DIM), jnp.float32),
    )(x_embed)

    idx, rs = pl.pallas_call(
        _topk_body,
        in_specs=[
            pl.BlockSpec((BATCH, EMBED_DIM), lambda: (0, 0)),
            pl.BlockSpec((POOL_SIZE, EMBED_DIM), lambda: (0, 0)),
        ],
        out_specs=[
            pl.BlockSpec((BATCH, TOP_K), lambda: (0, 0)),
            pl.BlockSpec((1, 1), lambda: (0, 0)),
        ],
        out_shape=[
            jax.ShapeDtypeStruct((BATCH, TOP_K), jnp.int32),
            jax.ShapeDtypeStruct((1, 1), jnp.float32),
        ],
    )(x_norm, prompt_key)

    j = jnp.arange(NUM_LAYERS * BATCH)
    d_in = (j // BATCH).astype(jnp.int32)
    l = jnp.arange(NUM_LAYERS, dtype=jnp.int32)
    g = ((l[:, None, None] * 2 + d_in[None, :, None]) * POOL_SIZE
         + idx[j % BATCH][None])
    g = g.reshape(NUM_WORKERS, NCHUNK, CHUNK).astype(jnp.int32)

    table = prompt.reshape(TABLE_ROWS, ROW_D)
    rows = _make_sc_gather()(table, g)
    batched_prompt = rows.reshape(
        NUM_LAYERS, BATCH, 2, TOP_K * LENGTH, NUM_HEADS, HEAD_DIM)
    return (batched_prompt, rs[0, 0])


# speedup vs baseline: 4.6903x; 4.6903x over previous
import jax
import jax.numpy as jnp
from jax import lax
from jax.experimental import pallas as pl
from jax.experimental.pallas import tpu as pltpu

NUM_LAYERS = 2
POOL_SIZE = 1000
LENGTH = 5
NUM_HEADS = 12
EMBED_DIM = 768
HEAD_DIM = EMBED_DIM // NUM_HEADS
TOP_K = 4
BATCH = 128
SEQ = 197

XB = 8
NCOL = 1024


def _xnorm_body(x_ref, o_ref):
    x = x_ref[...]
    m = jnp.mean(x, axis=0)
    ss = jnp.sum(m * m, axis=-1, keepdims=True)
    o_ref[...] = m * lax.rsqrt(jnp.maximum(ss, 1e-12))


def _topk_body(xn_ref, key_ref, idx_ref, rs_ref):
    xn = xn_ref[...]
    key = key_ref[...]
    ss = jnp.sum(key * key, axis=-1, keepdims=True)
    keyn = key * lax.rsqrt(jnp.maximum(ss, 1e-12))
    sim = lax.dot_general(xn, keyn, (((1,), (1,)), ((), ())),
                          preferred_element_type=jnp.float32)
    iota = lax.broadcasted_iota(jnp.int32, sim.shape, 1)
    total = jnp.float32(0.0)
    cols = []
    for _ in range(TOP_K):
        m = jnp.max(sim, axis=1, keepdims=True)
        am = jnp.min(jnp.where(sim == m, iota, jnp.int32(2**30)),
                     axis=1, keepdims=True)
        cols.append(am)
        total = total + jnp.sum(m)
        sim = jnp.where(iota == am, -jnp.inf, sim)
    idx_ref[...] = jnp.concatenate(cols, axis=1)
    rs_ref[...] = jnp.full((1, 1), total / BATCH, jnp.float32)


def _gather_mm_body(colidx_ref, p_ref, o_ref, oh0_ref, oh1_ref):
    l = pl.program_id(0)
    h = pl.program_id(1)

    @pl.when(jnp.logical_and(l == 0, h == 0))
    def _build_onehot():
        piota = lax.broadcasted_iota(jnp.int32, (POOL_SIZE, NCOL), 0)
        c0 = colidx_ref[0, :][None, :]
        c1 = colidx_ref[1, :][None, :]
        oh0_ref[...] = (piota == c0).astype(jnp.bfloat16)
        oh1_ref[...] = (piota == c1).astype(jnp.bfloat16)

    x = p_ref[...]
    a = x.reshape(2, LENGTH * HEAD_DIM, POOL_SIZE).astype(jnp.bfloat16)
    dn = (((1,), (0,)), ((), ()))
    out = (lax.dot_general(a[0], oh0_ref[...], dn,
                           preferred_element_type=jnp.float32)
           + lax.dot_general(a[1], oh1_ref[...], dn,
                             preferred_element_type=jnp.float32))
    for j in range(2 * TOP_K):
        o_ref[0, j, :, 0, :, :] = out[:, j * BATCH:(j + 1) * BATCH].reshape(
            LENGTH, HEAD_DIM, BATCH)


def kernel(x_embed, prompt, prompt_key):
    xt = jnp.transpose(x_embed, (1, 0, 2))
    x_norm = pl.pallas_call(
        _xnorm_body,
        grid=(BATCH // XB,),
        in_specs=[pl.BlockSpec((SEQ, XB, EMBED_DIM), lambda i: (0, i, 0))],
        out_specs=pl.BlockSpec((XB, EMBED_DIM), lambda i: (i, 0)),
        out_shape=jax.ShapeDtypeStruct((BATCH, EMBED_DIM), jnp.float32),
    )(xt)

    idx, rs = pl.pallas_call(
        _topk_body,
        in_specs=[
            pl.BlockSpec((BATCH, EMBED_DIM), lambda: (0, 0)),
            pl.BlockSpec((POOL_SIZE, EMBED_DIM), lambda: (0, 0)),
        ],
        out_specs=[
            pl.BlockSpec((BATCH, TOP_K), lambda: (0, 0)),
            pl.BlockSpec((1, 1), lambda: (0, 0)),
        ],
        out_shape=[
            jax.ShapeDtypeStruct((BATCH, TOP_K), jnp.int32),
            jax.ShapeDtypeStruct((1, 1), jnp.float32),
        ],
    )(x_norm, prompt_key)

    b = jnp.arange(BATCH)
    dp = jnp.arange(2)
    kk = jnp.arange(TOP_K)
    src = idx[2 * (b[None, None, :] % 64) + dp[:, None, None],
              kk[None, :, None]]
    dsel = (b // 64)[None, None, :]
    colidx = jnp.where(dsel == jnp.arange(2)[:, None, None, None],
                       src[None], 2000).reshape(2, NCOL).astype(jnp.int32)

    pt = jnp.transpose(prompt, (0, 1, 3, 4, 5, 2))
    a7 = pl.pallas_call(
        _gather_mm_body,
        grid=(NUM_LAYERS, NUM_HEADS),
        in_specs=[
            pl.BlockSpec((2, NCOL), lambda l, h: (0, 0)),
            pl.BlockSpec((1, 2, LENGTH, 1, HEAD_DIM, POOL_SIZE),
                         lambda l, h: (l, 0, 0, h, 0, 0)),
        ],
        out_specs=pl.BlockSpec((1, 2 * TOP_K, LENGTH, 1, HEAD_DIM, BATCH),
                               lambda l, h: (l, 0, 0, h, 0, 0)),
        out_shape=jax.ShapeDtypeStruct(
            (NUM_LAYERS, 2 * TOP_K, LENGTH, NUM_HEADS, HEAD_DIM, BATCH),
            jnp.float32),
        scratch_shapes=[
            pltpu.VMEM((POOL_SIZE, NCOL), jnp.bfloat16),
            pltpu.VMEM((POOL_SIZE, NCOL), jnp.bfloat16),
        ],
    )(colidx, pt)

    a6 = a7.reshape(NUM_LAYERS, 2, TOP_K * LENGTH, NUM_HEADS, HEAD_DIM, BATCH)
    batched_prompt = jnp.transpose(a6, (0, 5, 1, 2, 3, 4))
    return (batched_prompt, rs[0, 0])


# speedup vs baseline: 5.1776x; 1.1039x over previous
import jax
import jax.numpy as jnp
from jax import lax
from jax.experimental import pallas as pl
from jax.experimental.pallas import tpu as pltpu

NUM_LAYERS = 2
POOL_SIZE = 1000
LENGTH = 5
NUM_HEADS = 12
EMBED_DIM = 768
HEAD_DIM = EMBED_DIM // NUM_HEADS
TOP_K = 4
BATCH = 128
SEQ = 197

XB = 8
NXB = BATCH // XB
HB = 2
NCOL = 1024


def _sim_topk_body(x_ref, key_ref, idx_ref, rs_ref, xn_ref):
    i = pl.program_id(0)

    @pl.when(i < NXB)
    def _mean_norm():
        x = x_ref[...]
        m = jnp.mean(x, axis=0)
        ss = jnp.sum(m * m, axis=-1, keepdims=True)
        xn_ref[pl.ds(i * XB, XB), :] = m * lax.rsqrt(jnp.maximum(ss, 1e-12))

    @pl.when(i == NXB)
    def _topk():
        xn = xn_ref[...]
        key = key_ref[...]
        ss = jnp.sum(key * key, axis=-1, keepdims=True)
        keyn = key * lax.rsqrt(jnp.maximum(ss, 1e-12))
        sim = lax.dot_general(xn, keyn, (((1,), (1,)), ((), ())),
                              preferred_element_type=jnp.float32)
        iota = lax.broadcasted_iota(jnp.int32, sim.shape, 1)
        total = jnp.float32(0.0)
        cols = []
        for _ in range(TOP_K):
            m = jnp.max(sim, axis=1, keepdims=True)
            am = jnp.min(jnp.where(sim == m, iota, jnp.int32(2**30)),
                         axis=1, keepdims=True)
            cols.append(am)
            total = total + jnp.sum(m)
            sim = jnp.where(iota == am, -jnp.inf, sim)
        idx_ref[...] = jnp.concatenate(cols, axis=1)
        rs_ref[...] = jnp.full((1, 1), total / BATCH, jnp.float32)


def _gather_mm_body(colidx_ref, p_ref, o_ref, oh0_ref, oh1_ref):
    l = pl.program_id(0)
    h = pl.program_id(1)

    @pl.when(jnp.logical_and(l == 0, h == 0))
    def _build_onehot():
        piota = lax.broadcasted_iota(jnp.int32, (POOL_SIZE, NCOL), 0)
        c0 = colidx_ref[0, :][None, :]
        c1 = colidx_ref[1, :][None, :]
        oh0_ref[...] = (piota == c0).astype(jnp.bfloat16)
        oh1_ref[...] = (piota == c1).astype(jnp.bfloat16)

    x = p_ref[...]
    a = x.reshape(2, LENGTH * HB * HEAD_DIM, POOL_SIZE).astype(jnp.bfloat16)
    dn = (((1,), (0,)), ((), ()))
    out = (lax.dot_general(a[0], oh0_ref[...], dn,
                           preferred_element_type=jnp.float32)
           + lax.dot_general(a[1], oh1_ref[...], dn,
                             preferred_element_type=jnp.float32))
    for j in range(2 * TOP_K):
        o_ref[0, j, :, :, :, :] = out[:, j * BATCH:(j + 1) * BATCH].reshape(
            LENGTH, HB, HEAD_DIM, BATCH)


def kernel(x_embed, prompt, prompt_key):
    xt = jnp.transpose(x_embed, (1, 0, 2))
    idx, rs = pl.pallas_call(
        _sim_topk_body,
        grid=(NXB + 1,),
        in_specs=[
            pl.BlockSpec((SEQ, XB, EMBED_DIM),
                         lambda i: (0, jnp.minimum(i, NXB - 1), 0)),
            pl.BlockSpec((POOL_SIZE, EMBED_DIM), lambda i: (0, 0)),
        ],
        out_specs=[
            pl.BlockSpec((BATCH, TOP_K), lambda i: (0, 0)),
            pl.BlockSpec((1, 1), lambda i: (0, 0)),
        ],
        out_shape=[
            jax.ShapeDtypeStruct((BATCH, TOP_K), jnp.int32),
            jax.ShapeDtypeStruct((1, 1), jnp.float32),
        ],
        scratch_shapes=[pltpu.VMEM((BATCH, EMBED_DIM), jnp.float32)],
    )(xt, prompt_key)

    b = jnp.arange(BATCH)
    dp = jnp.arange(2)
    kk = jnp.arange(TOP_K)
    src = idx[2 * (b[None, None, :] % 64) + dp[:, None, None],
              kk[None, :, None]]
    dsel = (b // 64)[None, None, :]
    colidx = jnp.where(dsel == jnp.arange(2)[:, None, None, None],
                       src[None], 2000).reshape(2, NCOL).astype(jnp.int32)

    pt = jnp.transpose(prompt, (0, 1, 3, 4, 5, 2))
    a7 = pl.pallas_call(
        _gather_mm_body,
        grid=(NUM_LAYERS, NUM_HEADS // HB),
        in_specs=[
            pl.BlockSpec((2, NCOL), lambda l, h: (0, 0)),
            pl.BlockSpec((1, 2, LENGTH, HB, HEAD_DIM, POOL_SIZE),
                         lambda l, h: (l, 0, 0, h, 0, 0)),
        ],
        out_specs=pl.BlockSpec((1, 2 * TOP_K, LENGTH, HB, HEAD_DIM, BATCH),
                               lambda l, h: (l, 0, 0, h, 0, 0)),
        out_shape=jax.ShapeDtypeStruct(
            (NUM_LAYERS, 2 * TOP_K, LENGTH, NUM_HEADS, HEAD_DIM, BATCH),
            jnp.float32),
        scratch_shapes=[
            pltpu.VMEM((POOL_SIZE, NCOL), jnp.bfloat16),
            pltpu.VMEM((POOL_SIZE, NCOL), jnp.bfloat16),
        ],
    )(colidx, pt)

    a6 = a7.reshape(NUM_LAYERS, 2, TOP_K * LENGTH, NUM_HEADS, HEAD_DIM, BATCH)
    batched_prompt = jnp.transpose(a6, (0, 5, 1, 2, 3, 4))
    return (batched_prompt, rs[0, 0])


# speedup vs baseline: 5.2502x; 1.0140x over previous
import jax
import jax.numpy as jnp
from jax import lax
from jax.experimental import pallas as pl
from jax.experimental.pallas import tpu as pltpu

NUM_LAYERS = 2
POOL_SIZE = 1000
LENGTH = 5
NUM_HEADS = 12
EMBED_DIM = 768
HEAD_DIM = EMBED_DIM // NUM_HEADS
TOP_K = 4
BATCH = 128
SEQ = 197

XB = 8
NXB = BATCH // XB
HB = 4
NCOL = 1024


def _sim_topk_body(x_ref, key_ref, idx_ref, rs_ref, xn_ref):
    i = pl.program_id(0)

    @pl.when(i < NXB)
    def _mean_norm():
        x = x_ref[...]
        m = jnp.mean(x, axis=0)
        ss = jnp.sum(m * m, axis=-1, keepdims=True)
        xn_ref[pl.ds(i * XB, XB), :] = m * lax.rsqrt(jnp.maximum(ss, 1e-12))

    @pl.when(i == NXB)
    def _topk():
        xn = xn_ref[...]
        key = key_ref[...]
        ss = jnp.sum(key * key, axis=-1, keepdims=True)
        keyn = key * lax.rsqrt(jnp.maximum(ss, 1e-12))
        sim = lax.dot_general(xn, keyn, (((1,), (1,)), ((), ())),
                              preferred_element_type=jnp.float32)
        iota = lax.broadcasted_iota(jnp.int32, sim.shape, 1)
        total = jnp.float32(0.0)
        cols = []
        for _ in range(TOP_K):
            m = jnp.max(sim, axis=1, keepdims=True)
            am = jnp.min(jnp.where(sim == m, iota, jnp.int32(2**30)),
                         axis=1, keepdims=True)
            cols.append(am)
            total = total + jnp.sum(m)
            sim = jnp.where(iota == am, -jnp.inf, sim)
        idx_ref[...] = jnp.concatenate(cols, axis=1)
        rs_ref[...] = jnp.full((1, 1), total / BATCH, jnp.float32)


def _gather_mm_body(colidx_ref, p_ref, o_ref, oh0_ref, oh1_ref):
    l = pl.program_id(0)
    h = pl.program_id(1)

    @pl.when(jnp.logical_and(l == 0, h == 0))
    def _build_onehot():
        piota = lax.broadcasted_iota(jnp.int32, (POOL_SIZE, NCOL), 0)
        c0 = colidx_ref[0, :][None, :]
        c1 = colidx_ref[1, :][None, :]
        oh0_ref[...] = (piota == c0).astype(jnp.float32)
        oh1_ref[...] = (piota == c1).astype(jnp.float32)

    x = p_ref[...]
    a = x.reshape(2, LENGTH * HB * HEAD_DIM, POOL_SIZE)
    dn = (((1,), (0,)), ((), ()))
    out = (lax.dot_general(a[0], oh0_ref[...], dn,
                           preferred_element_type=jnp.float32)
           + lax.dot_general(a[1], oh1_ref[...], dn,
                             preferred_element_type=jnp.float32))
    for j in range(2 * TOP_K):
        o_ref[0, j, :, :, :, :] = out[:, j * BATCH:(j + 1) * BATCH].reshape(
            LENGTH, HB, HEAD_DIM, BATCH)


def kernel(x_embed, prompt, prompt_key):
    xt = jnp.transpose(x_embed, (1, 0, 2))
    idx, rs = pl.pallas_call(
        _sim_topk_body,
        grid=(NXB + 1,),
        in_specs=[
            pl.BlockSpec((SEQ, XB, EMBED_DIM),
                         lambda i: (0, jnp.minimum(i, NXB - 1), 0)),
            pl.BlockSpec((POOL_SIZE, EMBED_DIM), lambda i: (0, 0)),
        ],
        out_specs=[
            pl.BlockSpec((BATCH, TOP_K), lambda i: (0, 0)),
            pl.BlockSpec((1, 1), lambda i: (0, 0)),
        ],
        out_shape=[
            jax.ShapeDtypeStruct((BATCH, TOP_K), jnp.int32),
            jax.ShapeDtypeStruct((1, 1), jnp.float32),
        ],
        scratch_shapes=[pltpu.VMEM((BATCH, EMBED_DIM), jnp.float32)],
    )(xt, prompt_key)

    b = jnp.arange(BATCH)
    dp = jnp.arange(2)
    kk = jnp.arange(TOP_K)
    src = idx[2 * (b[None, None, :] % 64) + dp[:, None, None],
              kk[None, :, None]]
    dsel = (b // 64)[None, None, :]
    colidx = jnp.where(dsel == jnp.arange(2)[:, None, None, None],
                       src[None], 2000).reshape(2, NCOL).astype(jnp.int32)

    pt = jnp.transpose(prompt, (0, 1, 3, 4, 5, 2))
    a7 = pl.pallas_call(
        _gather_mm_body,
        grid=(NUM_LAYERS, NUM_HEADS // HB),
        in_specs=[
            pl.BlockSpec((2, NCOL), lambda l, h: (0, 0)),
            pl.BlockSpec((1, 2, LENGTH, HB, HEAD_DIM, POOL_SIZE),
                         lambda l, h: (l, 0, 0, h, 0, 0)),
        ],
        out_specs=pl.BlockSpec((1, 2 * TOP_K, LENGTH, HB, HEAD_DIM, BATCH),
                               lambda l, h: (l, 0, 0, h, 0, 0)),
        out_shape=jax.ShapeDtypeStruct(
            (NUM_LAYERS, 2 * TOP_K, LENGTH, NUM_HEADS, HEAD_DIM, BATCH),
            jnp.float32),
        scratch_shapes=[
            pltpu.VMEM((POOL_SIZE, NCOL), jnp.float32),
            pltpu.VMEM((POOL_SIZE, NCOL), jnp.float32),
        ],
    )(colidx, pt)

    a6 = a7.reshape(NUM_LAYERS, 2, TOP_K * LENGTH, NUM_HEADS, HEAD_DIM, BATCH)
    batched_prompt = jnp.transpose(a6, (0, 5, 1, 2, 3, 4))
    return (batched_prompt, rs[0, 0])


# speedup vs baseline: 6.0130x; 1.1453x over previous
import jax
import jax.numpy as jnp
from jax import lax
from jax.experimental import pallas as pl
from jax.experimental.pallas import tpu as pltpu

NUM_LAYERS = 2
POOL_SIZE = 1000
LENGTH = 5
NUM_HEADS = 12
EMBED_DIM = 768
HEAD_DIM = EMBED_DIM // NUM_HEADS
TOP_K = 4
BATCH = 128
SEQ = 197

XB = 8
NXB = BATCH // XB
HB = 4
NHB = NUM_HEADS // HB
NCOL = 1024
NGA = NXB + 1


def _body(x_ref, key_ref, p_ref, rs_ref, o_ref, xn_ref, oh0_ref, oh1_ref):
    i = pl.program_id(0)

    @pl.when(i < NXB)
    def _mean_norm():
        x = x_ref[...]
        m = jnp.mean(x, axis=0)
        ss = jnp.sum(m * m, axis=-1, keepdims=True)
        xn_ref[pl.ds(i * XB, XB), :] = m * lax.rsqrt(jnp.maximum(ss, 1e-12))

    @pl.when(i == NXB)
    def _topk():
        xn = xn_ref[...]
        key = key_ref[...]
        ss = jnp.sum(key * key, axis=-1, keepdims=True)
        keyn = key * lax.rsqrt(jnp.maximum(ss, 1e-12))
        sim = lax.dot_general(xn, keyn, (((1,), (1,)), ((), ())),
                              preferred_element_type=jnp.float32)
        iota = lax.broadcasted_iota(jnp.int32, sim.shape, 1)
        total = jnp.float32(0.0)
        masks = []
        for _ in range(TOP_K):
            m = jnp.max(sim, axis=1, keepdims=True)
            am = jnp.min(jnp.where(sim == m, iota, jnp.int32(2**30)),
                         axis=1, keepdims=True)
            hit = iota == am
            masks.append(hit.astype(jnp.float32))
            total = total + jnp.sum(m)
            sim = jnp.where(hit, -jnp.inf, sim)
        rs_ref[...] = jnp.full((1, 1), total / BATCH, jnp.float32)

        bi = lax.broadcasted_iota(jnp.int32, (BATCH, BATCH), 0)
        bo = lax.broadcasted_iota(jnp.int32, (BATCH, BATCH), 1)
        dn = (((0,), (0,)), ((), ()))
        for d, oh_ref in ((0, oh0_ref), (1, oh1_ref)):
            for dp in range(2):
                t = jnp.logical_and(bi == 2 * (bo % 64) + dp,
                                    bo // 64 == d).astype(jnp.float32)
                for k in range(TOP_K):
                    j = dp * TOP_K + k
                    oh_ref[:, pl.ds(j * BATCH, BATCH)] = lax.dot_general(
                        masks[k], t, dn, preferred_element_type=jnp.float32)

    @pl.when(i > NXB)
    def _gather_mm():
        x = p_ref[...]
        a = x.reshape(2, LENGTH * HB * HEAD_DIM, POOL_SIZE)
        dn = (((1,), (0,)), ((), ()))
        out = (lax.dot_general(a[0], oh0_ref[...], dn,
                               preferred_element_type=jnp.float32)
               + lax.dot_general(a[1], oh1_ref[...], dn,
                                 preferred_element_type=jnp.float32))
        for j in range(2 * TOP_K):
            o_ref[0, j, :, :, :, :] = out[:, j * BATCH:(j + 1) * BATCH].reshape(
                LENGTH, HB, HEAD_DIM, BATCH)


def kernel(x_embed, prompt, prompt_key):
    xt = jnp.transpose(x_embed, (1, 0, 2))
    pt = jnp.transpose(prompt, (0, 1, 3, 4, 5, 2))

    def _pt_map(i):
        g = jnp.maximum(i - NGA, 0)
        return (g // NHB, 0, 0, g % NHB, 0, 0)

    rs, a7 = pl.pallas_call(
        _body,
        grid=(NGA + NUM_LAYERS * NHB,),
        in_specs=[
            pl.BlockSpec((SEQ, XB, EMBED_DIM),
                         lambda i: (0, jnp.minimum(i, NXB - 1), 0)),
            pl.BlockSpec((POOL_SIZE, EMBED_DIM), lambda i: (0, 0)),
            pl.BlockSpec((1, 2, LENGTH, HB, HEAD_DIM, POOL_SIZE), _pt_map),
        ],
        out_specs=[
            pl.BlockSpec((1, 1), lambda i: (0, 0)),
            pl.BlockSpec((1, 2 * TOP_K, LENGTH, HB, HEAD_DIM, BATCH), _pt_map),
        ],
        out_shape=[
            jax.ShapeDtypeStruct((1, 1), jnp.float32),
            jax.ShapeDtypeStruct(
                (NUM_LAYERS, 2 * TOP_K, LENGTH, NUM_HEADS, HEAD_DIM, BATCH),
                jnp.float32),
        ],
        scratch_shapes=[
            pltpu.VMEM((BATCH, EMBED_DIM), jnp.float32),
            pltpu.VMEM((POOL_SIZE, NCOL), jnp.float32),
            pltpu.VMEM((POOL_SIZE, NCOL), jnp.float32),
        ],
    )(xt, prompt_key, pt)

    a6 = a7.reshape(NUM_LAYERS, 2, TOP_K * LENGTH, NUM_HEADS, HEAD_DIM, BATCH)
    batched_prompt = jnp.transpose(a6, (0, 5, 1, 2, 3, 4))
    return (batched_prompt, rs[0, 0])


# speedup vs baseline: 6.7561x; 1.1236x over previous
import jax
import jax.numpy as jnp
from jax import lax
from jax.experimental import pallas as pl
from jax.experimental.pallas import tpu as pltpu

NUM_LAYERS = 2
POOL_SIZE = 1000
LENGTH = 5
NUM_HEADS = 12
EMBED_DIM = 768
HEAD_DIM = EMBED_DIM // NUM_HEADS
TOP_K = 4
BATCH = 128
SEQ = 197

XB = 8
NXB = BATCH // XB
HB = 4
NHB = NUM_HEADS // HB
NCOL = 1024
NGA = NXB + 1


def _body(x_ref, key_ref, p_ref, rs_ref, o_ref, xn_ref, oh_ref):
    i = pl.program_id(0)

    @pl.when(i < NXB)
    def _mean_norm():
        x = x_ref[...]
        m = jnp.mean(x, axis=0)
        ss = jnp.sum(m * m, axis=-1, keepdims=True)
        xn_ref[pl.ds(i * XB, XB), :] = m * lax.rsqrt(jnp.maximum(ss, 1e-12))

    @pl.when(i == NXB)
    def _topk():
        xn = xn_ref[...]
        key = key_ref[...]
        ss = jnp.sum(key * key, axis=-1, keepdims=True)
        keyn = key * lax.rsqrt(jnp.maximum(ss, 1e-12))
        sim = lax.dot_general(xn, keyn, (((1,), (1,)), ((), ())),
                              preferred_element_type=jnp.float32)
        iota = lax.broadcasted_iota(jnp.int32, sim.shape, 1)
        total = jnp.float32(0.0)
        masks = []
        for _ in range(TOP_K):
            m = jnp.max(sim, axis=1, keepdims=True)
            am = jnp.min(jnp.where(sim == m, iota, jnp.int32(2**30)),
                         axis=1, keepdims=True)
            hit = iota == am
            masks.append(hit.astype(jnp.float32))
            total = total + jnp.sum(m)
            sim = jnp.where(hit, -jnp.inf, sim)
        rs_ref[...] = jnp.full((1, 1), total / BATCH, jnp.float32)

        bi = lax.broadcasted_iota(jnp.int32, (BATCH, 64), 0)
        co = lax.broadcasted_iota(jnp.int32, (BATCH, 64), 1)
        dn = (((0,), (0,)), ((), ()))
        for dp in range(2):
            t = (bi == 2 * co + dp).astype(jnp.float32)
            for k in range(TOP_K):
                j = dp * TOP_K + k
                oh_ref[:, pl.ds(j * 64, 64)] = lax.dot_general(
                    masks[k], t, dn, preferred_element_type=jnp.float32)

    @pl.when(i > NXB)
    def _gather_mm():
        x = p_ref[...]
        a = x.reshape(2, LENGTH * HB * HEAD_DIM, POOL_SIZE)
        dn = (((1,), (0,)), ((), ()))
        oh = oh_ref[...]
        for d in range(2):
            out = lax.dot_general(a[d], oh, dn,
                                  preferred_element_type=jnp.float32)
            for j in range(2 * TOP_K):
                o_ref[0, j, :, :, :, pl.ds(d * 64, 64)] = out[
                    :, j * 64:(j + 1) * 64].reshape(LENGTH, HB, HEAD_DIM, 64)


def kernel(x_embed, prompt, prompt_key):
    xt = jnp.transpose(x_embed, (1, 0, 2))
    pt = jnp.transpose(prompt, (0, 1, 3, 4, 5, 2))

    def _pt_map(i):
        g = jnp.maximum(i - NGA, 0)
        return (g // NHB, 0, 0, g % NHB, 0, 0)

    rs, a7 = pl.pallas_call(
        _body,
        grid=(NGA + NUM_LAYERS * NHB,),
        in_specs=[
            pl.BlockSpec((SEQ, XB, EMBED_DIM),
                         lambda i: (0, jnp.minimum(i, NXB - 1), 0)),
            pl.BlockSpec((POOL_SIZE, EMBED_DIM), lambda i: (0, 0)),
            pl.BlockSpec((1, 2, LENGTH, HB, HEAD_DIM, POOL_SIZE), _pt_map),
        ],
        out_specs=[
            pl.BlockSpec((1, 1), lambda i: (0, 0)),
            pl.BlockSpec((1, 2 * TOP_K, LENGTH, HB, HEAD_DIM, BATCH), _pt_map),
        ],
        out_shape=[
            jax.ShapeDtypeStruct((1, 1), jnp.float32),
            jax.ShapeDtypeStruct(
                (NUM_LAYERS, 2 * TOP_K, LENGTH, NUM_HEADS, HEAD_DIM, BATCH),
                jnp.float32),
        ],
        scratch_shapes=[
            pltpu.VMEM((BATCH, EMBED_DIM), jnp.float32),
            pltpu.VMEM((POOL_SIZE, 8 * 64), jnp.float32),
        ],
    )(xt, prompt_key, pt)

    a6 = a7.reshape(NUM_LAYERS, 2, TOP_K * LENGTH, NUM_HEADS, HEAD_DIM, BATCH)
    batched_prompt = jnp.transpose(a6, (0, 5, 1, 2, 3, 4))
    return (batched_prompt, rs[0, 0])


# speedup vs baseline: 6.7829x; 1.0040x over previous
import jax
import jax.numpy as jnp
from jax import lax
from jax.experimental import pallas as pl
from jax.experimental.pallas import tpu as pltpu

NUM_LAYERS = 2
POOL_SIZE = 1000
LENGTH = 5
NUM_HEADS = 12
EMBED_DIM = 768
HEAD_DIM = EMBED_DIM // NUM_HEADS
TOP_K = 4
BATCH = 128
SEQ = 197

XB = 16
NXB = BATCH // XB
HB = 4
NHB = NUM_HEADS // HB
NCOL = 1024
NGA = NXB + 1


def _body(x_ref, key_ref, p_ref, rs_ref, o_ref, xn_ref, oh_ref):
    i = pl.program_id(0)

    @pl.when(i < NXB)
    def _mean_norm():
        x = x_ref[...]
        m = jnp.mean(x, axis=0)
        ss = jnp.sum(m * m, axis=-1, keepdims=True)
        xn_ref[pl.ds(i * XB, XB), :] = m * lax.rsqrt(jnp.maximum(ss, 1e-12))

    @pl.when(i == NXB)
    def _topk():
        xn = xn_ref[...]
        key = key_ref[...]
        ss = jnp.sum(key * key, axis=-1, keepdims=True)
        keyn = key * lax.rsqrt(jnp.maximum(ss, 1e-12))
        sim = lax.dot_general(xn, keyn, (((1,), (1,)), ((), ())),
                              preferred_element_type=jnp.float32)
        iota = lax.broadcasted_iota(jnp.int32, sim.shape, 1)
        total = jnp.float32(0.0)
        masks = []
        for _ in range(TOP_K):
            m = jnp.max(sim, axis=1, keepdims=True)
            am = jnp.min(jnp.where(sim == m, iota, jnp.int32(2**30)),
                         axis=1, keepdims=True)
            hit = iota == am
            masks.append(hit.astype(jnp.float32))
            total = total + jnp.sum(m)
            sim = jnp.where(hit, -jnp.inf, sim)
        rs_ref[...] = jnp.full((1, 1), total / BATCH, jnp.float32)

        bi = lax.broadcasted_iota(jnp.int32, (BATCH, 64), 0)
        co = lax.broadcasted_iota(jnp.int32, (BATCH, 64), 1)
        dn = (((0,), (0,)), ((), ()))
        for dp in range(2):
            t = (bi == 2 * co + dp).astype(jnp.float32)
            for k in range(TOP_K):
                j = dp * TOP_K + k
                oh_ref[:, pl.ds(j * 64, 64)] = lax.dot_general(
                    masks[k], t, dn, preferred_element_type=jnp.float32)

    @pl.when(i > NXB)
    def _gather_mm():
        x = p_ref[...]
        a = x.reshape(2, LENGTH * HB * HEAD_DIM, POOL_SIZE)
        dn = (((1,), (0,)), ((), ()))
        oh = oh_ref[...]
        for d in range(2):
            out = lax.dot_general(a[d], oh, dn,
                                  preferred_element_type=jnp.float32)
            for j in range(2 * TOP_K):
                o_ref[0, j, :, :, :, pl.ds(d * 64, 64)] = out[
                    :, j * 64:(j + 1) * 64].reshape(LENGTH, HB, HEAD_DIM, 64)


def kernel(x_embed, prompt, prompt_key):
    xt = jnp.transpose(x_embed, (1, 0, 2))
    pt = jnp.transpose(prompt, (0, 1, 3, 4, 5, 2))

    def _pt_map(i):
        g = jnp.maximum(i - NGA, 0)
        return (g // NHB, 0, 0, g % NHB, 0, 0)

    rs, a7 = pl.pallas_call(
        _body,
        grid=(NGA + NUM_LAYERS * NHB,),
        in_specs=[
            pl.BlockSpec((SEQ, XB, EMBED_DIM),
                         lambda i: (0, jnp.minimum(i, NXB - 1), 0)),
            pl.BlockSpec((POOL_SIZE, EMBED_DIM), lambda i: (0, 0)),
            pl.BlockSpec((1, 2, LENGTH, HB, HEAD_DIM, POOL_SIZE), _pt_map),
        ],
        out_specs=[
            pl.BlockSpec((1, 1), lambda i: (0, 0)),
            pl.BlockSpec((1, 2 * TOP_K, LENGTH, HB, HEAD_DIM, BATCH), _pt_map),
        ],
        out_shape=[
            jax.ShapeDtypeStruct((1, 1), jnp.float32),
            jax.ShapeDtypeStruct(
                (NUM_LAYERS, 2 * TOP_K, LENGTH, NUM_HEADS, HEAD_DIM, BATCH),
                jnp.float32),
        ],
        scratch_shapes=[
            pltpu.VMEM((BATCH, EMBED_DIM), jnp.float32),
            pltpu.VMEM((POOL_SIZE, 8 * 64), jnp.float32),
        ],
    )(xt, prompt_key, pt)

    a6 = a7.reshape(NUM_LAYERS, 2, TOP_K * LENGTH, NUM_HEADS, HEAD_DIM, BATCH)
    batched_prompt = jnp.transpose(a6, (0, 5, 1, 2, 3, 4))
    return (batched_prompt, rs[0, 0])
